# baseline (speedup 1.0000x reference)
import dataclasses
import functools

import jax
import jax.numpy as jnp
from jax import lax
from jax.experimental import pallas as pl
from jax.experimental.pallas import tpu as pltpu
from jax.experimental.pallas import tpu_sc as plsc

_N = 10000
_E = 320000
_D = 128
_H = 128
_C = 64

_NC = 2
_NS = 16
_NW = _NC * _NS
_EW = _E // _NW
_CHE = 128
_ET = _E // _NS
_CPT = 160
_EPT = _CPT * _CHE
_NP = 10240
_NH = 5120
_ACC_R = 5248
_ZRT = _ACC_R // _NS
_DPT = _NH // _NS

_mesh = plsc.VectorSubcoreMesh(core_axis_name="c", subcore_axis_name="s")

_sc_params = pltpu.CompilerParams()
if "needs_layout_passes" in pltpu.CompilerParams.__dataclass_fields__:
    _sc_params = dataclasses.replace(_sc_params, needs_layout_passes=False)


def _sc_deg_body(dstf_hbm, out_hbm, dstv, hist):
    c = lax.axis_index("c")
    s = lax.axis_index("s")
    wid = c * _NS + s
    pltpu.sync_copy(dstf_hbm.at[pl.ds(wid * _EW, _EW)], dstv)
    z16 = jnp.zeros((16,), jnp.float32)

    @pl.loop(0, _NP, step=16)
    def _(i):
        hist[pl.ds(i, 16)] = z16

    ones16 = jnp.ones((16,), jnp.float32)

    @pl.loop(0, _EW, step=16)
    def _(j):
        idx = dstv[pl.ds(j, 16)]
        plsc.addupdate_scatter(hist, [idx], ones16)

    pltpu.sync_copy(hist, out_hbm.at[wid])


_sc_deg = pl.kernel(
    _sc_deg_body,
    out_type=jax.ShapeDtypeStruct((_NW, _NP), jnp.float32),
    mesh=_mesh,
    scratch_types=[
        pltpu.VMEM((_EW,), jnp.int32),
        pltpu.VMEM((_NP,), jnp.float32),
    ],
    compiler_params=_sc_params,
)


_CCAP = _EPT + 512


def _sc_agg_body(g_hbm, pck_hbm, out_hbm, pckc,
                 sa, sb, sc_, sd, da, db, dc, dd, rows_a, rows_b, rows_c,
                 rows_d, acc, sem_a, sem_b, sem_c, sem_d):
    c = lax.axis_index("c")
    s = lax.axis_index("s")
    base = c * _NH
    pltpu.sync_copy(pck_hbm.at[pl.ds(s * _EPT, _EPT)], pckc.at[pl.ds(0, _EPT)])

    z16 = jnp.zeros((16,), jnp.float32)

    @pl.loop(0, _CHE)
    def _(r):
        @pl.loop(0, _H, step=16)
        def _(col):
            rows_a[r, pl.ds(col, 16)] = z16

    pltpu.sync_copy(rows_a, acc.at[pl.ds(s * _ZRT, _CHE)])
    pltpu.sync_copy(rows_a, acc.at[pl.ds(s * _ZRT + _CHE, _CHE)])
    pltpu.sync_copy(rows_a.at[pl.ds(0, _ZRT - 2 * _CHE)],
                    acc.at[pl.ds(s * _ZRT + 2 * _CHE, _ZRT - 2 * _CHE)])

    def _cbody(i, off):
        p = pckc[pl.ds(i * 16, 16)]
        local = (p >> 16) - base
        ok = (local >= 0) & (local < _NH)
        repacked = (p & 0xFFFF) | (local << 16)
        plsc.store_compressed(pckc.at[pl.ds(off, 16)], repacked, mask=ok)
        pc = plsc.all_reduce_population_count(ok)
        return off + jnp.max(pc)

    off = lax.fori_loop(0, _EPT // 16, _cbody, jnp.int32(0))

    offp = ((off + 511) // 512) * 512
    tvec = (_NH + lax.iota(jnp.int32, 16)) << 16
    tmask = jnp.ones((16,), jnp.bool_)

    @pl.loop(0, 32)
    def _(g):
        plsc.store_compressed(pckc.at[pl.ds(off + g * 16, 16)], tvec,
                              mask=tmask)

    plsc.subcore_barrier()

    def _fill(sref, dref, o):
        @pl.loop(0, _CHE, step=16)
        def _(k):
            p = pckc[pl.ds(o + k, 16)]
            sref[pl.ds(k, 16)] = p & 0xFFFF
            dref[pl.ds(k, 16)] = p >> 16

    @pl.when(offp > 0)
    def _():
        _fill(sa, da, 0)
        pltpu.async_copy(g_hbm.at[sa], rows_a, sem_a)
        _fill(sb, db, _CHE)
        pltpu.async_copy(g_hbm.at[sb], rows_b, sem_b)

    @pl.loop(0, _CPT // 4)
    def _(t):
        o = t * 512

        @pl.when(o < offp)
        def _():
            _fill(sc_, dc, o + 2 * _CHE)
            pltpu.async_copy(g_hbm.at[sc_], rows_c, sem_c)
            pltpu.make_async_copy(g_hbm.at[sa], rows_a, sem_a).wait()
            pltpu.sync_copy(rows_a, acc.at[da], add=True)
            _fill(sd, dd, o + 3 * _CHE)
            pltpu.async_copy(g_hbm.at[sd], rows_d, sem_d)
            pltpu.make_async_copy(g_hbm.at[sb], rows_b, sem_b).wait()
            pltpu.sync_copy(rows_b, acc.at[db], add=True)

            @pl.when(o + 512 < offp)
            def _():
                _fill(sa, da, o + 512)
                pltpu.async_copy(g_hbm.at[sa], rows_a, sem_a)

            pltpu.make_async_copy(g_hbm.at[sc_], rows_c, sem_c).wait()
            pltpu.sync_copy(rows_c, acc.at[dc], add=True)

            @pl.when(o + 512 < offp)
            def _():
                _fill(sb, db, o + 512 + _CHE)
                pltpu.async_copy(g_hbm.at[sb], rows_b, sem_b)

            pltpu.make_async_copy(g_hbm.at[sd], rows_d, sem_d).wait()
            pltpu.sync_copy(rows_d, acc.at[dd], add=True)

    plsc.subcore_barrier()
    pltpu.sync_copy(acc.at[pl.ds(s * _DPT, _DPT)],
                    out_hbm.at[c, pl.ds(s * _DPT, _DPT)])


_sc_agg = pl.kernel(
    _sc_agg_body,
    out_type=jax.ShapeDtypeStruct((_NC, _NH, _H), jnp.float32),
    mesh=_mesh,
    scratch_types=[
        pltpu.VMEM((_CCAP,), jnp.int32),
        pltpu.VMEM((_CHE,), jnp.int32),
        pltpu.VMEM((_CHE,), jnp.int32),
        pltpu.VMEM((_CHE,), jnp.int32),
        pltpu.VMEM((_CHE,), jnp.int32),
        pltpu.VMEM((_CHE,), jnp.int32),
        pltpu.VMEM((_CHE,), jnp.int32),
        pltpu.VMEM((_CHE,), jnp.int32),
        pltpu.VMEM((_CHE,), jnp.int32),
        pltpu.VMEM((_CHE, _H), jnp.float32),
        pltpu.VMEM((_CHE, _H), jnp.float32),
        pltpu.VMEM((_CHE, _H), jnp.float32),
        pltpu.VMEM((_CHE, _H), jnp.float32),
        pltpu.VMEM_SHARED((_ACC_R, _H), jnp.float32),
        pltpu.SemaphoreType.DMA,
        pltpu.SemaphoreType.DMA,
        pltpu.SemaphoreType.DMA,
        pltpu.SemaphoreType.DMA,
    ],
    compiler_params=_sc_params,
)


def _agg_edges(g, pck):
    p = _sc_agg(g, pck)
    return p.reshape(_NC * _NH, _H)


_CDIMS = (((1,), (1,)), ((), ()))


def _dinv_from(ht_blk):
    deg = jnp.sum(ht_blk, axis=1, keepdims=True) + 1.0
    return lax.rsqrt(deg)


def _tc1_body(x_ref, ht_ref, w0_ref, b0_ref, w1_ref, h0_ref, g1s_ref):
    dinv = _dinv_from(ht_ref[...])
    h0 = jnp.maximum(
        lax.dot_general(x_ref[...], w0_ref[...], _CDIMS,
                        preferred_element_type=jnp.float32) + b0_ref[...], 0.0)
    h0_ref[...] = h0
    g1 = lax.dot_general(h0, w1_ref[...], _CDIMS,
                         preferred_element_type=jnp.float32)
    g1s_ref[...] = dinv * g1


def _tc2_body(p_ref, g1s_ref, h0_ref, ht_ref, ww_ref, wb_ref, w2_ref,
              m1_ref, g2s_ref):
    dinv = _dinv_from(ht_ref[...])
    agg = dinv * (p_ref[...] + g1s_ref[...])
    h1 = jnp.maximum(
        lax.dot_general(agg, ww_ref[...], _CDIMS,
                        preferred_element_type=jnp.float32) + wb_ref[...], 0.0)
    m1_ref[...] = jnp.maximum(h0_ref[...], h1)
    g2s_ref[...] = dinv * lax.dot_general(h1, w2_ref[...], _CDIMS,
                                          preferred_element_type=jnp.float32)


def _tc3_body(p_ref, g2s_ref, m1_ref, ht_ref, ww_ref, wb_ref, fw_ref, fb_ref,
              ow_ref, ob_ref, out_ref):
    dinv = _dinv_from(ht_ref[...])
    agg = dinv * (p_ref[...] + g2s_ref[...])
    h2 = jnp.maximum(
        lax.dot_general(agg, ww_ref[...], _CDIMS,
                        preferred_element_type=jnp.float32) + wb_ref[...], 0.0)
    m = jnp.maximum(m1_ref[...], h2)
    hf = lax.dot_general(m, fw_ref[...], _CDIMS,
                         preferred_element_type=jnp.float32) + fb_ref[...]
    out_ref[...] = lax.dot_general(hf, ow_ref[...], _CDIMS,
                                   preferred_element_type=jnp.float32) + ob_ref[...]


_B = 2000


def _full(shape):
    return pl.BlockSpec(shape, lambda i: tuple(0 for _ in shape))


def _rows(w):
    return pl.BlockSpec((_B, w), lambda i: (i, 0))


def _tc1(x, histsT, w0, b0, w1):
    return pl.pallas_call(
        _tc1_body,
        grid=(_N // _B,),
        in_specs=[_rows(_D), _rows(_NW), _full((_H, _D)), _full((1, _H)),
                  _full((_H, _H))],
        out_specs=[_rows(_H), _rows(_H)],
        out_shape=[jax.ShapeDtypeStruct((_N, _H), jnp.float32),
                   jax.ShapeDtypeStruct((_N, _H), jnp.float32)],
    )(x, histsT, w0, b0, w1)


def _tc2(p, g1s, h0, histsT, ww, wb, w2):
    return pl.pallas_call(
        _tc2_body,
        grid=(_N // _B,),
        in_specs=[_rows(_H), _rows(_H), _rows(_H), _rows(_NW),
                  _full((_H, _H)), _full((1, _H)), _full((_H, _H))],
        out_specs=[_rows(_H), _rows(_H)],
        out_shape=[jax.ShapeDtypeStruct((_N, _H), jnp.float32),
                   jax.ShapeDtypeStruct((_N, _H), jnp.float32)],
    )(p, g1s, h0, histsT, ww, wb, w2)


def _tc3(p, g2s, m1, histsT, ww, wb, fw, fb, ow, ob):
    return pl.pallas_call(
        _tc3_body,
        grid=(_N // _B,),
        in_specs=[_rows(_H), _rows(_H), _rows(_H), _rows(_NW),
                  _full((_H, _H)), _full((1, _H)), _full((_H, _H)),
                  _full((1, _H)), _full((_C, _H)), _full((1, _C))],
        out_specs=_rows(_C),
        out_shape=jax.ShapeDtypeStruct((_N, _C), jnp.float32),
    )(p, g2s, m1, histsT, ww, wb, fw, fb, ow, ob)


def kernel(x, edge_index, fc0_w, fc0_b, conv_w1, conv_w2, W_w, W_b,
           fcout_w, fcout_b, outlin_w, outlin_b):
    pck = (edge_index[0] | (edge_index[1] << 16)).reshape(_NS, _ET)
    ppad = jnp.full((_NS, _EPT - _ET), 0x3FFF << 16, jnp.int32)
    pckp = jnp.concatenate([pck, ppad], axis=1).reshape(_NS * _EPT)
    dstf = edge_index[1]

    hists = _sc_deg(dstf)
    histsT = hists.T

    h0, g1s = _tc1(x, histsT, fc0_w, fc0_b.reshape(1, _H), conv_w1)
    p1 = _agg_edges(g1s, pckp)
    m1, g2s = _tc2(p1, g1s, h0, histsT, W_w, W_b.reshape(1, _H), conv_w2)
    p2 = _agg_edges(g2s, pckp)
    out = _tc3(p2, g2s, m1, histsT, W_w, W_b.reshape(1, _H),
               fcout_w, fcout_b.reshape(1, _H),
               outlin_w, outlin_b.reshape(1, _C))
    return out


# speedup vs baseline: 1.5582x; 1.5582x over previous
import dataclasses
import functools

import jax
import jax.numpy as jnp
from jax import lax
from jax.experimental import pallas as pl
from jax.experimental.pallas import tpu as pltpu
from jax.experimental.pallas import tpu_sc as plsc

_N = 10000
_E = 320000
_D = 128
_H = 128
_C = 64

_NC = 2
_NS = 16
_NW = _NC * _NS
_EW = _E // _NW
_CHE = 128
_ET = _E // _NS
_CPT = 160
_EPT = _CPT * _CHE
_NP = 10240
_NH = 5120
_ACC_R = 5248
_ZRT = _ACC_R // _NS
_DPT = _NH // _NS

_mesh = plsc.VectorSubcoreMesh(core_axis_name="c", subcore_axis_name="s")

_sc_params = pltpu.CompilerParams()
if "needs_layout_passes" in pltpu.CompilerParams.__dataclass_fields__:
    _sc_params = dataclasses.replace(_sc_params, needs_layout_passes=False)


def _sc_deg_body(dstf_hbm, out_hbm, dstv, hist):
    c = lax.axis_index("c")
    s = lax.axis_index("s")
    wid = c * _NS + s
    pltpu.sync_copy(dstf_hbm.at[pl.ds(wid * _EW, _EW)], dstv)
    z16 = jnp.zeros((16,), jnp.float32)

    @pl.loop(0, _NP, step=16)
    def _(i):
        hist[pl.ds(i, 16)] = z16

    ones16 = jnp.ones((16,), jnp.float32)

    @pl.loop(0, _EW, step=16)
    def _(j):
        idx = dstv[pl.ds(j, 16)]
        plsc.addupdate_scatter(hist, [idx], ones16)

    pltpu.sync_copy(hist, out_hbm.at[wid])


_sc_deg = pl.kernel(
    _sc_deg_body,
    out_type=jax.ShapeDtypeStruct((_NW, _NP), jnp.float32),
    mesh=_mesh,
    scratch_types=[
        pltpu.VMEM((_EW,), jnp.int32),
        pltpu.VMEM((_NP,), jnp.float32),
    ],
    compiler_params=_sc_params,
)


_CCAP = _EPT + 256


def _sc_agg_body(g_hbm, pck_hbm, out_hbm, pckc,
                 sa, sb, da, db, rows_a, rows_b, acc, sem_a, sem_b):
    c = lax.axis_index("c")
    s = lax.axis_index("s")
    base = c * _NH
    pltpu.sync_copy(pck_hbm.at[pl.ds(s * _EPT, _EPT)], pckc.at[pl.ds(0, _EPT)])

    z16 = jnp.zeros((16,), jnp.float32)

    @pl.loop(0, _CHE)
    def _(r):
        @pl.loop(0, _H, step=16)
        def _(col):
            rows_a[r, pl.ds(col, 16)] = z16

    pltpu.sync_copy(rows_a, acc.at[pl.ds(s * _ZRT, _CHE)])
    pltpu.sync_copy(rows_a, acc.at[pl.ds(s * _ZRT + _CHE, _CHE)])
    pltpu.sync_copy(rows_a.at[pl.ds(0, _ZRT - 2 * _CHE)],
                    acc.at[pl.ds(s * _ZRT + 2 * _CHE, _ZRT - 2 * _CHE)])

    def _cbody(i, off):
        p = pckc[pl.ds(i * 16, 16)]
        local = (p >> 16) - base
        ok = (local >= 0) & (local < _NH)
        repacked = (p & 0xFFFF) | (local << 16)
        plsc.store_compressed(pckc.at[pl.ds(off, 16)], repacked, mask=ok)
        pc = plsc.all_reduce_population_count(ok)
        return off + jnp.max(pc)

    off = lax.fori_loop(0, _EPT // 16, _cbody, jnp.int32(0))

    offp = ((off + 255) // 256) * 256
    tvec = (_NH + lax.iota(jnp.int32, 16)) << 16
    tmask = jnp.ones((16,), jnp.bool_)

    @pl.loop(0, 16)
    def _(g):
        plsc.store_compressed(pckc.at[pl.ds(off + g * 16, 16)], tvec,
                              mask=tmask)

    plsc.subcore_barrier()

    def _fill(sref, dref, o):
        @pl.loop(0, _CHE, step=16)
        def _(k):
            p = pckc[pl.ds(o + k, 16)]
            sref[pl.ds(k, 16)] = p & 0xFFFF
            dref[pl.ds(k, 16)] = p >> 16

    @pl.when(offp > 0)
    def _():
        _fill(sa, da, 0)
        pltpu.async_copy(g_hbm.at[sa], rows_a, sem_a)

    @pl.loop(0, _CPT // 2)
    def _(t):
        o = t * 256

        @pl.when(o < offp)
        def _():
            _fill(sb, db, o + _CHE)
            cpb = pltpu.async_copy(g_hbm.at[sb], rows_b, sem_b)
            pltpu.make_async_copy(g_hbm.at[sa], rows_a, sem_a).wait()
            pltpu.sync_copy(rows_a, acc.at[da], add=True)

            @pl.when(o + 256 < offp)
            def _():
                _fill(sa, da, o + 256)
                pltpu.async_copy(g_hbm.at[sa], rows_a, sem_a)

            cpb.wait()
            pltpu.sync_copy(rows_b, acc.at[db], add=True)

    plsc.subcore_barrier()
    pltpu.sync_copy(acc.at[pl.ds(s * _DPT, _DPT)],
                    out_hbm.at[c, pl.ds(s * _DPT, _DPT)])


_sc_agg = pl.kernel(
    _sc_agg_body,
    out_type=jax.ShapeDtypeStruct((_NC, _NH, _H), jnp.float32),
    mesh=_mesh,
    scratch_types=[
        pltpu.VMEM((_CCAP,), jnp.int32),
        pltpu.VMEM((_CHE,), jnp.int32),
        pltpu.VMEM((_CHE,), jnp.int32),
        pltpu.VMEM((_CHE,), jnp.int32),
        pltpu.VMEM((_CHE,), jnp.int32),
        pltpu.VMEM((_CHE, _H), jnp.float32),
        pltpu.VMEM((_CHE, _H), jnp.float32),
        pltpu.VMEM_SHARED((_ACC_R, _H), jnp.float32),
        pltpu.SemaphoreType.DMA,
        pltpu.SemaphoreType.DMA,
    ],
    compiler_params=_sc_params,
)


def _agg_edges(g, pck):
    p = _sc_agg(g, pck)
    return p.reshape(_NC * _NH, _H)


_CDIMS = (((1,), (1,)), ((), ()))


def _dinv_from(ht_blk):
    deg = jnp.sum(ht_blk, axis=1, keepdims=True) + 1.0
    return lax.rsqrt(deg)


def _tc1_body(x_ref, ht_ref, w0_ref, b0_ref, w1_ref, h0_ref, g1s_ref):
    dinv = _dinv_from(ht_ref[...])
    h0 = jnp.maximum(
        lax.dot_general(x_ref[...], w0_ref[...], _CDIMS,
                        preferred_element_type=jnp.float32) + b0_ref[...], 0.0)
    h0_ref[...] = h0
    g1 = lax.dot_general(h0, w1_ref[...], _CDIMS,
                         preferred_element_type=jnp.float32)
    g1s_ref[...] = dinv * g1


def _tc2_body(p_ref, g1s_ref, h0_ref, ht_ref, ww_ref, wb_ref, w2_ref,
              m1_ref, g2s_ref):
    dinv = _dinv_from(ht_ref[...])
    agg = dinv * (p_ref[...] + g1s_ref[...])
    h1 = jnp.maximum(
        lax.dot_general(agg, ww_ref[...], _CDIMS,
                        preferred_element_type=jnp.float32) + wb_ref[...], 0.0)
    m1_ref[...] = jnp.maximum(h0_ref[...], h1)
    g2s_ref[...] = dinv * lax.dot_general(h1, w2_ref[...], _CDIMS,
                                          preferred_element_type=jnp.float32)


def _tc3_body(p_ref, g2s_ref, m1_ref, ht_ref, ww_ref, wb_ref, fw_ref, fb_ref,
              ow_ref, ob_ref, out_ref):
    dinv = _dinv_from(ht_ref[...])
    agg = dinv * (p_ref[...] + g2s_ref[...])
    h2 = jnp.maximum(
        lax.dot_general(agg, ww_ref[...], _CDIMS,
                        preferred_element_type=jnp.float32) + wb_ref[...], 0.0)
    m = jnp.maximum(m1_ref[...], h2)
    hf = lax.dot_general(m, fw_ref[...], _CDIMS,
                         preferred_element_type=jnp.float32) + fb_ref[...]
    out_ref[...] = lax.dot_general(hf, ow_ref[...], _CDIMS,
                                   preferred_element_type=jnp.float32) + ob_ref[...]


_B = 5000


def _full(shape):
    return pl.BlockSpec(shape, lambda i: tuple(0 for _ in shape))


def _rows(w):
    return pl.BlockSpec((_B, w), lambda i: (i, 0))


def _tc1(x, histsT, w0, b0, w1):
    return pl.pallas_call(
        _tc1_body,
        grid=(_N // _B,),
        in_specs=[_rows(_D), _rows(_NW), _full((_H, _D)), _full((1, _H)),
                  _full((_H, _H))],
        out_specs=[_rows(_H), _rows(_H)],
        out_shape=[jax.ShapeDtypeStruct((_N, _H), jnp.float32),
                   jax.ShapeDtypeStruct((_N, _H), jnp.float32)],
    )(x, histsT, w0, b0, w1)


def _tc2(p, g1s, h0, histsT, ww, wb, w2):
    return pl.pallas_call(
        _tc2_body,
        grid=(_N // _B,),
        in_specs=[_rows(_H), _rows(_H), _rows(_H), _rows(_NW),
                  _full((_H, _H)), _full((1, _H)), _full((_H, _H))],
        out_specs=[_rows(_H), _rows(_H)],
        out_shape=[jax.ShapeDtypeStruct((_N, _H), jnp.float32),
                   jax.ShapeDtypeStruct((_N, _H), jnp.float32)],
    )(p, g1s, h0, histsT, ww, wb, w2)


def _tc3(p, g2s, m1, histsT, ww, wb, fw, fb, ow, ob):
    return pl.pallas_call(
        _tc3_body,
        grid=(_N // _B,),
        in_specs=[_rows(_H), _rows(_H), _rows(_H), _rows(_NW),
                  _full((_H, _H)), _full((1, _H)), _full((_H, _H)),
                  _full((1, _H)), _full((_C, _H)), _full((1, _C))],
        out_specs=_rows(_C),
        out_shape=jax.ShapeDtypeStruct((_N, _C), jnp.float32),
    )(p, g2s, m1, histsT, ww, wb, fw, fb, ow, ob)


def kernel(x, edge_index, fc0_w, fc0_b, conv_w1, conv_w2, W_w, W_b,
           fcout_w, fcout_b, outlin_w, outlin_b):
    pck = (edge_index[0] | (edge_index[1] << 16)).reshape(_NS, _ET)
    ppad = jnp.full((_NS, _EPT - _ET), 0x3FFF << 16, jnp.int32)
    pckp = jnp.concatenate([pck, ppad], axis=1).reshape(_NS * _EPT)
    dstf = edge_index[1]

    hists = _sc_deg(dstf)
    histsT = hists.T

    h0, g1s = _tc1(x, histsT, fc0_w, fc0_b.reshape(1, _H), conv_w1)
    p1 = _agg_edges(g1s, pckp)
    m1, g2s = _tc2(p1, g1s, h0, histsT, W_w, W_b.reshape(1, _H), conv_w2)
    p2 = _agg_edges(g2s, pckp)
    out = _tc3(p2, g2s, m1, histsT, W_w, W_b.reshape(1, _H),
               fcout_w, fcout_b.reshape(1, _H),
               outlin_w, outlin_b.reshape(1, _C))
    return out
